# baseline (speedup 1.0000x reference)
import jax
import jax.numpy as jnp
from jax.experimental import pallas as pl
from jax.experimental.pallas import tpu as pltpu

B, H, W = 32, 512, 512
H_CHUNK = 128
N_H = H // H_CHUNK


def _loss_body(mask_ref, p_ref, out_ref, acc_ref):
    h = pl.program_id(1)
    p = p_ref[...]
    log_p = jnp.maximum(jnp.log(p), -100.0)
    contrib = jnp.sum(jnp.where(mask_ref[...], -log_p, 0.0))

    @pl.when(h == 0)
    def _init():
        acc_ref[0] = 0.0

    acc_ref[0] += contrib

    @pl.when(h == N_H - 1)
    def _fin():
        out_ref[0, 0, 0] = acc_ref[0]


def kernel(pos_indicator, pred_confs):
    p = pred_confs.reshape(B, H, W)
    out = pl.pallas_call(
        _loss_body,
        grid=(B, N_H),
        in_specs=[
            pl.BlockSpec((1, H_CHUNK, W), lambda b, h: (b, h, 0)),
            pl.BlockSpec((1, H_CHUNK, W), lambda b, h: (b, h, 0)),
        ],
        out_specs=pl.BlockSpec(
            (1, 1, 1), lambda b, h: (b, 0, 0), memory_space=pltpu.SMEM
        ),
        out_shape=jax.ShapeDtypeStruct((B, 1, 1), jnp.float32),
        scratch_shapes=[pltpu.SMEM((1,), jnp.float32)],
        compiler_params=pltpu.CompilerParams(
            dimension_semantics=("parallel", "arbitrary"),
        ),
    )(pos_indicator, p)
    return out.reshape(B)


# speedup vs baseline: 1.4805x; 1.4805x over previous
import jax
import jax.numpy as jnp
import numpy as np
from jax.experimental import pallas as pl
from jax.experimental.pallas import tpu as pltpu

B, H, W = 32, 512, 512

_C64 = np.float32(2.0**64)
_LN2 = float(np.log(2.0))
_QMIN = np.float32(np.exp(-100.0) * 2.0**64)
_Q0 = np.float32(np.exp(-100.0 + 64.0 * _LN2))


def _loss_body(mask_ref, p_ref, out_ref):
    p = p_ref[...].reshape(H, W)
    mask = mask_ref[...].reshape(H, W)

    q = p * _C64
    t = jnp.where(q < _QMIN, _Q0, q)
    v = jnp.where(mask, t, _C64)

    bits = v.view(jnp.int32)
    e_sum = jnp.sum(bits >> 23)
    m = ((bits & 0x7FFFFF) | 0x3F800000).view(jnp.float32)
    gp = m.reshape(H // 32, 32, W)
    while gp.shape[1] > 1:
        half = gp.shape[1] // 2
        gp = gp[:, :half, :] * gp[:, half:, :]
    m_sum = jnp.sum(jnp.log(gp[:, 0, :]))

    n = H * W
    out_ref[0, 0, 0] = -(_LN2 * (e_sum - 191 * n).astype(jnp.float32) + m_sum)


def kernel(pos_indicator, pred_confs):
    p = pred_confs.reshape(B, H, W)
    out = pl.pallas_call(
        _loss_body,
        grid=(B,),
        in_specs=[
            pl.BlockSpec((1, H, W), lambda b: (b, 0, 0)),
            pl.BlockSpec((1, H, W), lambda b: (b, 0, 0)),
        ],
        out_specs=pl.BlockSpec(
            (1, 1, 1), lambda b: (b, 0, 0), memory_space=pltpu.SMEM
        ),
        out_shape=jax.ShapeDtypeStruct((B, 1, 1), jnp.float32),
        compiler_params=pltpu.CompilerParams(
            dimension_semantics=("arbitrary",),
        ),
    )(pos_indicator, p)
    return out.reshape(B)
